# baseline (speedup 1.0000x reference)
import jax
import jax.numpy as jnp
import numpy as np
from jax import lax
from jax.experimental import pallas as pl

_N = 2
_C = 256
_A = 9
_K = 80
_NUM_CONVS = 4
_STRIDES = (8, 16, 32, 64, 128)
_LEVEL_HW = ((100, 152), (50, 76), (25, 38), (13, 19), (7, 10))
_IMG_H, _IMG_W = 800.0, 1216.0
_SCORE_THRESH = 0.05
_TOPK = 1000
_NMS_THRESH = 0.5
_MAX_DET = 100
_SCALE_CLAMP = float(np.log(1000.0 / 16.0))
_DN = ('NCHW', 'OIHW', 'NCHW')


def _cell_anchors(stride):
    out = []
    for k in range(3):
        size = 4.0 * stride * (2.0 ** (k / 3.0))
        area = size * size
        for ar in (0.5, 1.0, 2.0):
            w = (area / ar) ** 0.5
            h = ar * w
            out.append([-w / 2, -h / 2, w / 2, h / 2])
    return np.asarray(out, np.float32)


def _make_anchors(H, W, stride):
    cell = _cell_anchors(stride)
    sx = (np.arange(W, dtype=np.float32) + 0.5) * stride
    sy = (np.arange(H, dtype=np.float32) + 0.5) * stride
    gy, gx = np.meshgrid(sy, sx, indexing='ij')
    shifts = np.stack([gx, gy, gx, gy], -1)
    anch = (shifts[:, :, None, :] + cell[None, None]).reshape(-1, 4)
    return jnp.asarray(anch)


def _conv3(x, w, b):
    y = lax.conv_general_dilated(x, w, (1, 1), 'SAME', dimension_numbers=_DN)
    return y + b[None, :, None, None]


def _subnet(x, ws, bs):
    for i in range(_NUM_CONVS):
        x = jax.nn.relu(_conv3(x, ws[i], bs[i]))
    return x


def _permute_NHWA_K(t, k):
    n, _, h, w = t.shape
    return t.reshape(n, -1, k, h, w).transpose(0, 3, 4, 1, 2).reshape(n, -1, k)


def _decode(deltas, anchors):
    wa = anchors[:, 2] - anchors[:, 0]
    ha = anchors[:, 3] - anchors[:, 1]
    cxa = anchors[:, 0] + 0.5 * wa
    cya = anchors[:, 1] + 0.5 * ha
    dx, dy = deltas[:, 0], deltas[:, 1]
    dw = jnp.minimum(deltas[:, 2], _SCALE_CLAMP)
    dh = jnp.minimum(deltas[:, 3], _SCALE_CLAMP)
    cx = dx * wa + cxa
    cy = dy * ha + cya
    w = jnp.exp(dw) * wa
    h = jnp.exp(dh) * ha
    x1 = jnp.clip(cx - 0.5 * w, 0.0, _IMG_W)
    y1 = jnp.clip(cy - 0.5 * h, 0.0, _IMG_H)
    x2 = jnp.clip(cx + 0.5 * w, 0.0, _IMG_W)
    y2 = jnp.clip(cy + 0.5 * h, 0.0, _IMG_H)
    return jnp.stack([x1, y1, x2, y2], -1)


def _nms_single(boxes, scores, classes):
    off = classes.astype(boxes.dtype) * 4096.0
    b = boxes + off[:, None]
    area = (b[:, 2] - b[:, 0]) * (b[:, 3] - b[:, 1])
    idxs = jnp.arange(b.shape[0])

    def step(s, _):
        i = jnp.argmax(s)
        sc = s[i]
        valid = sc > 0.0
        bi = b[i]
        ix1 = jnp.maximum(b[:, 0], bi[0])
        iy1 = jnp.maximum(b[:, 1], bi[1])
        ix2 = jnp.minimum(b[:, 2], bi[2])
        iy2 = jnp.minimum(b[:, 3], bi[3])
        inter = jnp.maximum(ix2 - ix1, 0.0) * jnp.maximum(iy2 - iy1, 0.0)
        iou = inter / (area + area[i] - inter + 1e-9)
        kill = (iou >= _NMS_THRESH) | (idxs == i)
        s2 = jnp.where(valid & kill, -1.0, s)
        out = (jnp.where(valid, boxes[i], 0.0),
               jnp.where(valid, sc, 0.0),
               jnp.where(valid, classes[i], -1))
        return s2, out

    _, (kb, ks, kc) = lax.scan(step, scores, None, length=_MAX_DET)
    return kb, ks, kc


def _pallas_identity(x):
    def body(x_ref, o_ref):
        o_ref[...] = x_ref[...]
    return pl.pallas_call(
        body,
        out_shape=jax.ShapeDtypeStruct(x.shape, x.dtype),
    )(x)


def kernel(feat0, feat1, feat2, feat3, feat4, cls_w, cls_b, box_w, box_b,
           score_w, score_b, pred_w, pred_b):
    feats = [feat0, feat1, feat2, feat3, feat4]
    all_b, all_s, all_c = [], [], []
    for feat, (H, W), stride in zip(feats, _LEVEL_HW, _STRIDES):
        logits = _conv3(_subnet(feat, cls_w, cls_b), score_w, score_b)
        deltas = _conv3(_subnet(feat, box_w, box_b), pred_w, pred_b)
        lg = _permute_NHWA_K(logits, _K)
        dl = _permute_NHWA_K(deltas, 4)
        anchors = _make_anchors(H, W, stride)
        k = min(_TOPK, H * W * _A * _K)

        def per_img(lg1, dl1):
            s = jax.nn.sigmoid(lg1).reshape(-1)
            vals, idx = lax.top_k(s, k)
            a_idx = idx // _K
            cls = idx % _K
            bx = _decode(dl1[a_idx], anchors[a_idx])
            return bx, vals, cls

        bx, vals, cls = jax.vmap(per_img)(lg, dl)
        all_b.append(bx); all_s.append(vals); all_c.append(cls)

    boxes = jnp.concatenate(all_b, 1)
    scores = jnp.concatenate(all_s, 1)
    classes = jnp.concatenate(all_c, 1)
    scores = jnp.where(scores > _SCORE_THRESH, scores, -1.0)
    dets = jnp.zeros((_N, _MAX_DET, 5), jnp.float32) + (
        jnp.sum(boxes) + jnp.sum(scores) + jnp.sum(classes))
    kc = jnp.zeros((_N, _MAX_DET), jnp.int32)
    dets = _pallas_identity(dets)
    return dets, kc


def _unused_full(feat0, feat1, feat2, feat3, feat4, cls_w, cls_b, box_w, box_b,
                 score_w, score_b, pred_w, pred_b):
    feats = [feat0, feat1, feat2, feat3, feat4]
    all_b, all_s, all_c = [], [], []
    for feat, (H, W), stride in zip(feats, _LEVEL_HW, _STRIDES):
        logits = _conv3(_subnet(feat, cls_w, cls_b), score_w, score_b)
        deltas = _conv3(_subnet(feat, box_w, box_b), pred_w, pred_b)
        lg = _permute_NHWA_K(logits, _K)
        dl = _permute_NHWA_K(deltas, 4)
        anchors = _make_anchors(H, W, stride)
        k = min(_TOPK, H * W * _A * _K)

        def per_img(lg1, dl1):
            s = jax.nn.sigmoid(lg1).reshape(-1)
            vals, idx = lax.top_k(s, k)
            a_idx = idx // _K
            cls = idx % _K
            bx = _decode(dl1[a_idx], anchors[a_idx])
            return bx, vals, cls

        bx, vals, cls = jax.vmap(per_img)(lg, dl)
        all_b.append(bx); all_s.append(vals); all_c.append(cls)

    boxes = jnp.concatenate(all_b, 1)
    scores = jnp.concatenate(all_s, 1)
    classes = jnp.concatenate(all_c, 1)
    scores = jnp.where(scores > _SCORE_THRESH, scores, -1.0)
    kb, ks, kc = jax.vmap(_nms_single)(boxes, scores, classes)
    dets = jnp.concatenate([kb, ks[..., None]], -1)
    dets = _pallas_identity(dets)
    return dets, kc


# speedup vs baseline: 1.0226x; 1.0226x over previous
import jax
import jax.numpy as jnp
import numpy as np
from jax import lax
from jax.experimental import pallas as pl
from jax.experimental.pallas import tpu as pltpu

_N = 2
_C = 256
_A = 9
_K = 80
_NUM_CONVS = 4
_STRIDES = (8, 16, 32, 64, 128)
_LEVEL_HW = ((100, 152), (50, 76), (25, 38), (13, 19), (7, 10))
_IMG_H, _IMG_W = 800.0, 1216.0
_SCORE_THRESH = 0.05
_TOPK = 1000
_POOL = 1024
_NMS_THRESH = 0.5
_MAX_DET = 100
_SCALE_CLAMP = float(np.log(1000.0 / 16.0))
_DN = ('NCHW', 'OIHW', 'NCHW')

_NCAND = _TOPK * len(_LEVEL_HW)
_ROWS, _LANES = 40, 128
_PAD = _ROWS * _LANES


def _cell_anchors(stride):
    out = []
    for k in range(3):
        size = 4.0 * stride * (2.0 ** (k / 3.0))
        area = size * size
        for ar in (0.5, 1.0, 2.0):
            w = (area / ar) ** 0.5
            h = ar * w
            out.append([-w / 2, -h / 2, w / 2, h / 2])
    return np.asarray(out, np.float32)


def _conv3(x, w, b):
    y = lax.conv_general_dilated(x, w, (1, 1), 'SAME', dimension_numbers=_DN)
    return y + b[None, :, None, None]


def _subnet(x, ws, bs):
    for i in range(_NUM_CONVS):
        x = jax.nn.relu(_conv3(x, ws[i], bs[i]))
    return x


def _decode_nms_kernel(dl_ref, an_ref, sc_ref, cl_ref, o_ref):
    dx = dl_ref[0, 0]
    dy = dl_ref[0, 1]
    dw = dl_ref[0, 2]
    dh = dl_ref[0, 3]
    ax1 = an_ref[0, 0]
    ay1 = an_ref[0, 1]
    ax2 = an_ref[0, 2]
    ay2 = an_ref[0, 3]
    vals = sc_ref[0]
    cls = cl_ref[0]

    wa = ax2 - ax1
    ha = ay2 - ay1
    cxa = ax1 + 0.5 * wa
    cya = ay1 + 0.5 * ha
    dwc = jnp.minimum(dw, _SCALE_CLAMP)
    dhc = jnp.minimum(dh, _SCALE_CLAMP)
    cx = dx * wa + cxa
    cy = dy * ha + cya
    bw = jnp.exp(dwc) * wa
    bh = jnp.exp(dhc) * ha
    x1 = jnp.clip(cx - 0.5 * bw, 0.0, _IMG_W)
    y1 = jnp.clip(cy - 0.5 * bh, 0.0, _IMG_H)
    x2 = jnp.clip(cx + 0.5 * bw, 0.0, _IMG_W)
    y2 = jnp.clip(cy + 0.5 * bh, 0.0, _IMG_H)

    s0 = jnp.where(vals > _SCORE_THRESH, vals, -1.0)
    off = cls * 4096.0
    bx1 = x1 + off
    by1 = y1 + off
    bx2 = x2 + off
    by2 = y2 + off
    area = (bx2 - bx1) * (by2 - by1)

    iota_r = lax.broadcasted_iota(jnp.int32, (_ROWS, _LANES), 0)
    iota_l = lax.broadcasted_iota(jnp.int32, (_ROWS, _LANES), 1)
    flat_iota = iota_r * _LANES + iota_l
    out_rows = lax.broadcasted_iota(jnp.int32, (128, 128), 0)
    out_lane = lax.broadcasted_iota(jnp.int32, (128, 128), 1)

    def step(i, carry):
        s, out = carry
        sc = jnp.max(s)
        sel = jnp.min(jnp.where(s == sc, flat_iota, jnp.int32(1 << 30)))
        eq = flat_iota == sel
        valid = sc > 0.0

        def pick(v):
            return jnp.sum(jnp.where(eq, v, 0.0))

        x1i = pick(x1)
        y1i = pick(y1)
        x2i = pick(x2)
        y2i = pick(y2)
        ci = pick(cls)
        bx1i = pick(bx1)
        by1i = pick(by1)
        bx2i = pick(bx2)
        by2i = pick(by2)
        ai = pick(area)

        ix1 = jnp.maximum(bx1, bx1i)
        iy1 = jnp.maximum(by1, by1i)
        ix2 = jnp.minimum(bx2, bx2i)
        iy2 = jnp.minimum(by2, by2i)
        inter = jnp.maximum(ix2 - ix1, 0.0) * jnp.maximum(iy2 - iy1, 0.0)
        iou = inter / (area + ai - inter + 1e-9)
        kill = (iou >= _NMS_THRESH) | eq
        s2 = jnp.where(valid & kill, -1.0, s)

        vx1 = jnp.where(valid, x1i, 0.0)
        vy1 = jnp.where(valid, y1i, 0.0)
        vx2 = jnp.where(valid, x2i, 0.0)
        vy2 = jnp.where(valid, y2i, 0.0)
        vsc = jnp.where(valid, sc, 0.0)
        vcl = jnp.where(valid, ci, -1.0)
        rowv = (jnp.where(out_lane == 0, vx1, 0.0)
                + jnp.where(out_lane == 1, vy1, 0.0)
                + jnp.where(out_lane == 2, vx2, 0.0)
                + jnp.where(out_lane == 3, vy2, 0.0)
                + jnp.where(out_lane == 4, vsc, 0.0)
                + jnp.where(out_lane == 5, vcl, 0.0))
        out2 = jnp.where(out_rows == i, rowv, out)
        return s2, out2

    _, out = lax.fori_loop(0, _MAX_DET,
                           step, (s0, jnp.zeros((128, 128), jnp.float32)))
    o_ref[0] = out


def _postprocess(logits_list, deltas_list):
    all_vals, all_cls, all_dl, all_an = [], [], [], []
    for lv, ((H, W), stride) in enumerate(zip(_LEVEL_HW, _STRIDES)):
        HW = H * W
        logits = logits_list[lv]
        deltas = deltas_list[lv]
        flat = logits.reshape(_N, _A * _K * HW)
        lvals, idx = lax.top_k(flat, _POOL)
        svals = jax.nn.sigmoid(lvals)
        ch = idx // HW
        pos = idx % HW
        kcls = ch % _K
        a9 = ch // _K
        ref_idx = (pos * _A + a9) * _K + kcls
        nsort, _, pos_s, a9_s, kcls_s = lax.sort(
            (-svals, ref_idx, pos, a9, kcls), num_keys=2)
        vals = -nsort[:, :_TOPK]
        pos = pos_s[:, :_TOPK]
        a9 = a9_s[:, :_TOPK]
        kcls = kcls_s[:, :_TOPK]

        dflat = deltas.reshape(_N, _A * 4 * HW)
        base = a9 * (4 * HW) + pos
        didx = base[..., None] + (jnp.arange(4, dtype=idx.dtype) * HW)[None, None, :]
        dl = jnp.take_along_axis(
            dflat[:, :, None], didx.reshape(_N, -1, 1), axis=1
        ).reshape(_N, _TOPK, 4)

        cell = jnp.asarray(_cell_anchors(stride))
        ca = jnp.take(cell, a9, axis=0)
        wcol = pos % W
        hrow = pos // W
        sx = (wcol.astype(jnp.float32) + 0.5) * stride
        sy = (hrow.astype(jnp.float32) + 0.5) * stride
        shift = jnp.stack([sx, sy, sx, sy], axis=-1)
        an = shift + ca

        all_vals.append(vals)
        all_cls.append(kcls)
        all_dl.append(dl)
        all_an.append(an)

    vals = jnp.concatenate(all_vals, axis=1)
    cls = jnp.concatenate(all_cls, axis=1).astype(jnp.float32)
    dl = jnp.concatenate(all_dl, axis=1)
    an = jnp.concatenate(all_an, axis=1)

    pad = _PAD - _NCAND
    vals_p = jnp.pad(vals, ((0, 0), (0, pad)), constant_values=-1.0)
    cls_p = jnp.pad(cls, ((0, 0), (0, pad)))
    dl_p = jnp.pad(dl.transpose(0, 2, 1), ((0, 0), (0, 0), (0, pad)))
    an_p = jnp.pad(an.transpose(0, 2, 1), ((0, 0), (0, 0), (0, pad)))

    out = pl.pallas_call(
        _decode_nms_kernel,
        grid=(_N,),
        in_specs=[
            pl.BlockSpec((1, 4, _ROWS, _LANES), lambda n: (n, 0, 0, 0)),
            pl.BlockSpec((1, 4, _ROWS, _LANES), lambda n: (n, 0, 0, 0)),
            pl.BlockSpec((1, _ROWS, _LANES), lambda n: (n, 0, 0)),
            pl.BlockSpec((1, _ROWS, _LANES), lambda n: (n, 0, 0)),
        ],
        out_specs=pl.BlockSpec((1, 128, 128), lambda n: (n, 0, 0)),
        out_shape=jax.ShapeDtypeStruct((_N, 128, 128), jnp.float32),
        compiler_params=pltpu.CompilerParams(
            dimension_semantics=('parallel',),
        ),
    )(
        dl_p.reshape(_N, 4, _ROWS, _LANES),
        an_p.reshape(_N, 4, _ROWS, _LANES),
        vals_p.reshape(_N, _ROWS, _LANES),
        cls_p.reshape(_N, _ROWS, _LANES),
    )

    dets = out[:, :_MAX_DET, 0:5]
    kc = out[:, :_MAX_DET, 5].astype(jnp.int32)
    return dets, kc


def kernel(feat0, feat1, feat2, feat3, feat4, cls_w, cls_b, box_w, box_b,
           score_w, score_b, pred_w, pred_b):
    feats = [feat0, feat1, feat2, feat3, feat4]
    logits_list, deltas_list = [], []
    for feat in feats:
        logits_list.append(_conv3(_subnet(feat, cls_w, cls_b), score_w, score_b))
        deltas_list.append(_conv3(_subnet(feat, box_w, box_b), pred_w, pred_b))
    return _postprocess(logits_list, deltas_list)
